# baseline (speedup 1.0000x reference)
import functools

import jax
import jax.numpy as jnp
from jax import lax
from jax.experimental import pallas as pl
from jax.experimental.pallas import tpu as pltpu
from jax.experimental.pallas import tpu_sc as plsc

_NUM_CORES = 2
_NUM_SUBCORES = 16
_NUM_WORKERS = _NUM_CORES * _NUM_SUBCORES
_LANES = 16
_JB = 128
_NBUF = 4


@functools.partial(jax.jit, static_argnums=(2,))
def _gather_rows(idx_t, table, dim):
    seq, num_b = idx_t.shape
    b_per_w = num_b // _NUM_WORKERS
    jb_per_w = b_per_w // _JB
    n_blocks = seq * jb_per_w
    dim8 = dim // 8
    mesh = plsc.VectorSubcoreMesh(core_axis_name="c", subcore_axis_name="s")

    @functools.partial(
        pl.kernel,
        mesh=mesh,
        out_type=jax.ShapeDtypeStruct((seq, dim8, num_b // _JB, 8, _JB), jnp.float32),
        scratch_types=[
            pltpu.VMEM((seq, b_per_w), jnp.int32),
            [pltpu.VMEM((_JB, dim), jnp.float32) for _ in range(_NBUF)],
            [pltpu.VMEM((dim, _JB), jnp.float32) for _ in range(_NBUF)],
            [pltpu.SemaphoreType.DMA for _ in range(_NBUF)],
            [pltpu.SemaphoreType.DMA for _ in range(_NBUF)],
        ],
        compiler_params=pltpu.CompilerParams(
            use_tc_tiling_on_sc=False, needs_layout_passes=False
        ),
    )
    def gather_kernel(idx_hbm, table_hbm, out_hbm, idx_v, rows, trv, gsem, wsem):
        wid = lax.axis_index("s") * _NUM_CORES + lax.axis_index("c")
        pltpu.sync_copy(idx_hbm.at[:, pl.ds(wid * b_per_w, b_per_w)], idx_v)

        def fire(n, buf):
            s = n // jb_per_w
            b = n % jb_per_w
            pltpu.async_copy(
                table_hbm.at[idx_v.at[s, pl.ds(b * _JB, _JB)]],
                rows[buf],
                gsem[buf],
            )

        for b in range(_NBUF):
            fire(b, b)

        lane = lax.iota(jnp.int32, _LANES)
        row_ids = [j0 + lane for j0 in range(0, _JB, _LANES)]

        def body(g, carry):
            for buf in range(_NBUF):
                n = g * _NBUF + buf
                s = n // jb_per_w
                b = n % jb_per_w
                jb = wid * jb_per_w + b
                pltpu.make_async_copy(
                    table_hbm.at[idx_v.at[0, pl.ds(0, _JB)]],
                    rows[buf], gsem[buf],
                ).wait()
                @pl.when(n >= _NBUF)
                def _():
                    for db in range(dim8):
                        pltpu.make_async_copy(
                            trv[buf].at[pl.ds(db * 8, 8)],
                            out_hbm.at[0, 0, 0],
                            wsem[buf],
                        ).wait()

                def transpose_d(d, c, buf=buf):
                    col = jnp.broadcast_to(d, (_LANES,))
                    for k in range(_JB // _LANES):
                        vals = plsc.load_gather(rows[buf], [row_ids[k], col])
                        trv[buf][d, pl.ds(k * _LANES, _LANES)] = vals
                    return c

                lax.fori_loop(0, dim, transpose_d, 0)

                for db in range(dim8):
                    pltpu.async_copy(
                        trv[buf].at[pl.ds(db * 8, 8)], out_hbm.at[s, db, jb], wsem[buf]
                    )

                @pl.when(n + _NBUF < n_blocks)
                def _():
                    fire(n + _NBUF, buf)

            return carry

        lax.fori_loop(0, n_blocks // _NBUF, body, 0)

        for buf in range(_NBUF):
            for db in range(dim8):
                pltpu.make_async_copy(
                    trv[buf].at[pl.ds(db * 8, 8)], out_hbm.at[0, 0, 0], wsem[buf]
                ).wait()

    return gather_kernel(idx_t, table)


def kernel(token_ids, embedding_table):
    num_b, seq = token_ids.shape
    dim = embedding_table.shape[1]
    idx_t = token_ids.T.astype(jnp.int32)
    out5d = _gather_rows(idx_t, embedding_table, dim)
    return out5d.transpose(2, 4, 0, 1, 3).reshape(num_b, seq, dim)
